# baseline (speedup 1.0000x reference)
import functools

import jax
import jax.numpy as jnp
from jax import lax
from jax.experimental import pallas as pl
from jax.experimental.pallas import tpu as pltpu
from jax.experimental.pallas import tpu_sc as plsc

N = 10000
D = 128
E = 320000

NC = 2
NS = 16
NW = NC * NS

CHUNK = 64
CPT = 160
SG = 40
NSG = CPT // SG
NBUF = 4
DELTA = 1
NGRPS = SG // NBUF
EPT = CPT * CHUNK
EPAD = NW * EPT
ACC_N = 10240
ZRPT = ACC_N // NS
EPS = 1e-5

_mesh = plsc.VectorSubcoreMesh(core_axis_name="c", subcore_axis_name="s")


DEGW = 8


def _sc_deg_body(dst_hbm, deg_out, dstv, onesv, zbuf, acc, dsem):
    cid = lax.axis_index("c")
    sid = lax.axis_index("s")
    wid = cid * NS + sid

    pltpu.sync_copy(dst_hbm.at[wid], dstv)

    def _fill_ones(i, c):
        onesv[pl.ds(i * 16, 16)] = jnp.ones((16,), jnp.float32)
        return c
    lax.fori_loop(0, CHUNK // 16, _fill_ones, 0)

    @pl.when(sid == 0)
    def _init():
        def _z(i, c):
            zbuf[pl.ds(i * 16, 16)] = jnp.zeros((16,), jnp.float32)
            return c
        lax.fori_loop(0, ACC_N // 16, _z, 0)
        pltpu.sync_copy(zbuf, acc)

    plsc.subcore_barrier()

    def _chunk(j, c):
        pltpu.async_copy(onesv, acc.at[dstv.at[j]], dsem, add=True)

        @pl.when(j >= DEGW)
        def _w():
            pltpu.make_async_copy(onesv, acc.at[dstv.at[j - DEGW]],
                                  dsem).wait()
        return c
    lax.fori_loop(0, CPT, _chunk, 0)

    def _dw(j, c):
        pltpu.make_async_copy(onesv, acc.at[dstv.at[j]], dsem).wait()
        return c
    lax.fori_loop(CPT - DEGW, CPT, _dw, 0)

    plsc.subcore_barrier()

    @pl.when(sid == 0)
    def _out():
        pltpu.sync_copy(acc, deg_out.at[cid])


_sc_deg = pl.kernel(
    _sc_deg_body,
    out_type=jax.ShapeDtypeStruct((NC, ACC_N), jnp.float32),
    mesh=_mesh,
    scratch_types=[
        pltpu.VMEM((CPT, CHUNK), jnp.int32),
        pltpu.VMEM((CHUNK,), jnp.float32),
        pltpu.VMEM((ACC_N,), jnp.float32),
        pltpu.VMEM_SHARED((ACC_N,), jnp.float32),
        pltpu.SemaphoreType.DMA,
    ],
)


def _sc_agg_body(src_hbm, dst_hbm, hp_hbm, part_out, srcv, dstv,
                 r0, r1, r2, r3, acc, g0, g1, g2, g3, s0, s1, s2, s3):
    rows = (r0, r1, r2, r3)
    gsem = (g0, g1, g2, g3)
    ssem = (s0, s1, s2, s3)
    cid = lax.axis_index("c")
    sid = lax.axis_index("s")
    wid = cid * NS + sid

    def _zr(r, c):
        def _zc(k, c2):
            r0[r, pl.ds(k * 16, 16)] = jnp.zeros((16,), jnp.float32)
            return c2
        return lax.fori_loop(0, D // 16, _zc, c)
    lax.fori_loop(0, CHUNK, _zr, 0)

    zbase = sid * ZRPT
    for k in range(ZRPT // CHUNK):
        pltpu.sync_copy(r0, acc.at[pl.ds(zbase + k * CHUNK, CHUNK)])

    plsc.subcore_barrier()

    def _gather(j, b):
        pltpu.async_copy(hp_hbm.at[srcv.at[j]], rows[b], gsem[b])

    def _gather_wait(j, b):
        pltpu.make_async_copy(hp_hbm.at[srcv.at[j]], rows[b], gsem[b]).wait()

    def _scatter(j, b):
        pltpu.async_copy(rows[b], acc.at[dstv.at[j]], ssem[b], add=True)

    def _scatter_wait(j, b):
        pltpu.make_async_copy(rows[b], acc.at[dstv.at[j]], ssem[b]).wait()

    def _stage(t, c):
        pltpu.sync_copy(src_hbm.at[wid, pl.ds(t * SG, SG)], srcv)
        pltpu.sync_copy(dst_hbm.at[wid, pl.ds(t * SG, SG)], dstv)

        def _group(g, c2):
            for b in range(NBUF):
                j = g * NBUF + b

                @pl.when(g > 0)
                def _drain(b=b, j=j):
                    _scatter_wait(j - NBUF, b)
                _gather(j, b)

                bs = (b - DELTA) % NBUF
                js = j - DELTA

                @pl.when(js >= 0)
                def _scat(js=js, bs=bs):
                    _gather_wait(js, bs)
                    _scatter(js, bs)
            return c2
        lax.fori_loop(0, NGRPS, _group, 0)

        for r in range(DELTA):
            js = SG - DELTA + r
            bs = js % NBUF
            _gather_wait(js, bs)
            _scatter(js, bs)
        for b in range(NBUF):
            _scatter_wait(SG - NBUF + b, b)
        return c
    lax.fori_loop(0, NSG, _stage, 0)

    plsc.subcore_barrier()

    obase = sid * ZRPT
    pltpu.sync_copy(acc.at[pl.ds(obase, ZRPT)],
                    part_out.at[cid, pl.ds(obase, ZRPT)])


_sc_agg = pl.kernel(
    _sc_agg_body,
    out_type=jax.ShapeDtypeStruct((NC, ACC_N, D), jnp.float32),
    mesh=_mesh,
    scratch_types=[
        pltpu.VMEM((SG, CHUNK), jnp.int32),
        pltpu.VMEM((SG, CHUNK), jnp.int32),
        pltpu.VMEM((CHUNK, D), jnp.float32),
        pltpu.VMEM((CHUNK, D), jnp.float32),
        pltpu.VMEM((CHUNK, D), jnp.float32),
        pltpu.VMEM((CHUNK, D), jnp.float32),
        pltpu.VMEM_SHARED((ACC_N, D), jnp.float32),
        pltpu.SemaphoreType.DMA,
        pltpu.SemaphoreType.DMA,
        pltpu.SemaphoreType.DMA,
        pltpu.SemaphoreType.DMA,
        pltpu.SemaphoreType.DMA,
        pltpu.SemaphoreType.DMA,
        pltpu.SemaphoreType.DMA,
        pltpu.SemaphoreType.DMA,
    ],
)


def _tc_deg_body(degp_ref, dis_ref):
    deg = jnp.sum(degp_ref[...], axis=0, keepdims=True) + 1.0
    dis_ref[...] = lax.rsqrt(deg)


_tc_deg = pl.pallas_call(
    _tc_deg_body,
    out_shape=jax.ShapeDtypeStruct((1, ACC_N), jnp.float32),
)


def _tc_mm1_body(x_ref, w_ref, h_ref):
    h_ref[...] = jnp.dot(x_ref[...], w_ref[...],
                         preferred_element_type=jnp.float32)


_tc_mm1 = pl.pallas_call(
    _tc_mm1_body,
    out_shape=jax.ShapeDtypeStruct((N, D), jnp.float32),
)


def _tc_scale_body(h_ref, dis_ref, hp_ref):
    hp_ref[...] = h_ref[...] * dis_ref[...]


_tc_scale = pl.pallas_call(
    _tc_scale_body,
    out_shape=jax.ShapeDtypeStruct((N, D), jnp.float32),
)


def _bn_prelu(z, gamma, beta, alpha):
    mu = jnp.mean(z, axis=0, keepdims=True)
    xc = z - mu
    var = jnp.mean(xc * xc, axis=0, keepdims=True)
    bn = xc * lax.rsqrt(var + EPS) * gamma + beta
    return jnp.where(bn >= 0.0, bn, alpha * bn)


def _tc_mid_body(p_ref, hp_ref, dis_ref, b_ref, g_ref, be_ref, a_ref, w_ref,
                 out_ref):
    agg = p_ref[0, :N] + p_ref[1, :N] + hp_ref[...]
    z = agg * dis_ref[...] + b_ref[...]
    act = _bn_prelu(z, g_ref[...], be_ref[...], a_ref[...])
    h = jnp.dot(act, w_ref[...], preferred_element_type=jnp.float32)
    out_ref[...] = h * dis_ref[...]


_tc_mid = pl.pallas_call(
    _tc_mid_body,
    out_shape=jax.ShapeDtypeStruct((N, D), jnp.float32),
)


def _tc_post_body(p_ref, hp_ref, dis_ref, b_ref, g_ref, be_ref, a_ref,
                  out_ref):
    agg = p_ref[0, :N] + p_ref[1, :N] + hp_ref[...]
    z = agg * dis_ref[...] + b_ref[...]
    out_ref[...] = _bn_prelu(z, g_ref[...], be_ref[...], a_ref[...])


_tc_post = pl.pallas_call(
    _tc_post_body,
    out_shape=jax.ShapeDtypeStruct((N, D), jnp.float32),
)


def kernel(x, edge_index, W1, b1, gamma1, beta1, a1, W2, b2, gamma2, beta2, a2):
    src = edge_index[0].astype(jnp.int32)
    dst = edge_index[1].astype(jnp.int32)

    npad = EPAD - E
    pad_pos = jnp.arange(npad, dtype=jnp.int32)
    src_p = jnp.concatenate([src, pad_pos % N])
    dst_p = jnp.concatenate([dst, N + (pad_pos % (ACC_N - N))])
    srcr = src_p.reshape(NW, CPT, CHUNK)
    dstr = dst_p.reshape(NW, CPT, CHUNK)

    h1 = _tc_mm1(x, W1)
    degp = _sc_deg(dstr)
    dis_row = _tc_deg(degp)
    dis = dis_row[0, :N].reshape(N, 1)

    b1r, g1r, be1r = (v.reshape(1, D) for v in (b1, gamma1, beta1))
    b2r, g2r, be2r = (v.reshape(1, D) for v in (b2, gamma2, beta2))
    a1r = a1.reshape(1, 1)
    a2r = a2.reshape(1, 1)

    hp1 = _tc_scale(h1, dis)
    p1 = _sc_agg(srcr, dstr, hp1)
    hp2 = _tc_mid(p1, hp1, dis, b1r, g1r, be1r, a1r, W2)
    p2 = _sc_agg(srcr, dstr, hp2)
    out = _tc_post(p2, hp2, dis, b2r, g2r, be2r, a2r)
    return out


# speedup vs baseline: 1.1686x; 1.1686x over previous
import functools

import jax
import jax.numpy as jnp
from jax import lax
from jax.experimental import pallas as pl
from jax.experimental.pallas import tpu as pltpu
from jax.experimental.pallas import tpu_sc as plsc

N = 10000
D = 128
E = 320000

NC = 2
NS = 16
NW = NC * NS

CHUNK = 64
CPT = 160
SG = 40
NSG = CPT // SG
NBUF = 4
DELTA = 3
NGRPS = SG // NBUF
EPT = CPT * CHUNK
EPAD = NW * EPT
ACC_N = 10240
ZRPT = ACC_N // NS
EPS = 1e-5

_mesh = plsc.VectorSubcoreMesh(core_axis_name="c", subcore_axis_name="s")


DEGW = 8


def _sc_deg_body(dst_hbm, deg_out, dstv, onesv, zbuf, acc, dsem):
    cid = lax.axis_index("c")
    sid = lax.axis_index("s")
    wid = cid * NS + sid

    pltpu.sync_copy(dst_hbm.at[wid], dstv)

    def _fill_ones(i, c):
        onesv[pl.ds(i * 16, 16)] = jnp.ones((16,), jnp.float32)
        return c
    lax.fori_loop(0, CHUNK // 16, _fill_ones, 0)

    @pl.when(sid == 0)
    def _init():
        def _z(i, c):
            zbuf[pl.ds(i * 16, 16)] = jnp.zeros((16,), jnp.float32)
            return c
        lax.fori_loop(0, ACC_N // 16, _z, 0)
        pltpu.sync_copy(zbuf, acc)

    plsc.subcore_barrier()

    def _chunk(j, c):
        pltpu.async_copy(onesv, acc.at[dstv.at[j]], dsem, add=True)

        @pl.when(j >= DEGW)
        def _w():
            pltpu.make_async_copy(onesv, acc.at[dstv.at[j - DEGW]],
                                  dsem).wait()
        return c
    lax.fori_loop(0, CPT, _chunk, 0)

    def _dw(j, c):
        pltpu.make_async_copy(onesv, acc.at[dstv.at[j]], dsem).wait()
        return c
    lax.fori_loop(CPT - DEGW, CPT, _dw, 0)

    plsc.subcore_barrier()

    @pl.when(sid == 0)
    def _out():
        pltpu.sync_copy(acc, deg_out.at[cid])


_sc_deg = pl.kernel(
    _sc_deg_body,
    out_type=jax.ShapeDtypeStruct((NC, ACC_N), jnp.float32),
    mesh=_mesh,
    scratch_types=[
        pltpu.VMEM((CPT, CHUNK), jnp.int32),
        pltpu.VMEM((CHUNK,), jnp.float32),
        pltpu.VMEM((ACC_N,), jnp.float32),
        pltpu.VMEM_SHARED((ACC_N,), jnp.float32),
        pltpu.SemaphoreType.DMA,
    ],
)


def _sc_agg_body(src_hbm, dst_hbm, hp_hbm, part_out, srcv, dstv,
                 r0, r1, r2, r3, acc, g0, g1, g2, g3, s0, s1, s2, s3):
    rows = (r0, r1, r2, r3)
    gsem = (g0, g1, g2, g3)
    ssem = (s0, s1, s2, s3)
    cid = lax.axis_index("c")
    sid = lax.axis_index("s")
    wid = cid * NS + sid

    def _zr(r, c):
        def _zc(k, c2):
            r0[r, pl.ds(k * 16, 16)] = jnp.zeros((16,), jnp.float32)
            return c2
        return lax.fori_loop(0, D // 16, _zc, c)
    lax.fori_loop(0, CHUNK, _zr, 0)

    zbase = sid * ZRPT
    for k in range(ZRPT // CHUNK):
        pltpu.sync_copy(r0, acc.at[pl.ds(zbase + k * CHUNK, CHUNK)])

    plsc.subcore_barrier()

    def _gather(j, b):
        pltpu.async_copy(hp_hbm.at[srcv.at[j]], rows[b], gsem[b])

    def _gather_wait(j, b):
        pltpu.make_async_copy(hp_hbm.at[srcv.at[j]], rows[b], gsem[b]).wait()

    def _scatter(j, b):
        pltpu.async_copy(rows[b], acc.at[dstv.at[j]], ssem[b], add=True)

    def _scatter_wait(j, b):
        pltpu.make_async_copy(rows[b], acc.at[dstv.at[j]], ssem[b]).wait()

    def _stage(t, c):
        pltpu.sync_copy(src_hbm.at[wid, pl.ds(t * SG, SG)], srcv)
        pltpu.sync_copy(dst_hbm.at[wid, pl.ds(t * SG, SG)], dstv)

        def _group(g, c2):
            for b in range(NBUF):
                j = g * NBUF + b

                @pl.when(g > 0)
                def _drain(b=b, j=j):
                    _scatter_wait(j - NBUF, b)
                _gather(j, b)

                bs = (b - DELTA) % NBUF
                js = j - DELTA

                @pl.when(js >= 0)
                def _scat(js=js, bs=bs):
                    _gather_wait(js, bs)
                    _scatter(js, bs)
            return c2
        lax.fori_loop(0, NGRPS, _group, 0)

        for r in range(DELTA):
            js = SG - DELTA + r
            bs = js % NBUF
            _gather_wait(js, bs)
            _scatter(js, bs)
        for b in range(NBUF):
            _scatter_wait(SG - NBUF + b, b)
        return c
    lax.fori_loop(0, NSG, _stage, 0)

    plsc.subcore_barrier()

    obase = sid * ZRPT
    pltpu.sync_copy(acc.at[pl.ds(obase, ZRPT)],
                    part_out.at[cid, pl.ds(obase, ZRPT)])


_sc_agg = pl.kernel(
    _sc_agg_body,
    out_type=jax.ShapeDtypeStruct((NC, ACC_N, D), jnp.float32),
    mesh=_mesh,
    scratch_types=[
        pltpu.VMEM((SG, CHUNK), jnp.int32),
        pltpu.VMEM((SG, CHUNK), jnp.int32),
        pltpu.VMEM((CHUNK, D), jnp.float32),
        pltpu.VMEM((CHUNK, D), jnp.float32),
        pltpu.VMEM((CHUNK, D), jnp.float32),
        pltpu.VMEM((CHUNK, D), jnp.float32),
        pltpu.VMEM_SHARED((ACC_N, D), jnp.float32),
        pltpu.SemaphoreType.DMA,
        pltpu.SemaphoreType.DMA,
        pltpu.SemaphoreType.DMA,
        pltpu.SemaphoreType.DMA,
        pltpu.SemaphoreType.DMA,
        pltpu.SemaphoreType.DMA,
        pltpu.SemaphoreType.DMA,
        pltpu.SemaphoreType.DMA,
    ],
)


def _tc_deg_body(degp_ref, dis_ref):
    deg = jnp.sum(degp_ref[...], axis=0, keepdims=True) + 1.0
    dis_ref[...] = lax.rsqrt(deg)


_tc_deg = pl.pallas_call(
    _tc_deg_body,
    out_shape=jax.ShapeDtypeStruct((1, ACC_N), jnp.float32),
)


def _tc_mm1_body(x_ref, w_ref, h_ref):
    h_ref[...] = jnp.dot(x_ref[...], w_ref[...],
                         preferred_element_type=jnp.float32)


_tc_mm1 = pl.pallas_call(
    _tc_mm1_body,
    out_shape=jax.ShapeDtypeStruct((N, D), jnp.float32),
)


def _tc_scale_body(h_ref, dis_ref, hp_ref):
    hp_ref[...] = h_ref[...] * dis_ref[...]


_tc_scale = pl.pallas_call(
    _tc_scale_body,
    out_shape=jax.ShapeDtypeStruct((N, D), jnp.float32),
)


def _bn_prelu(z, gamma, beta, alpha):
    mu = jnp.mean(z, axis=0, keepdims=True)
    xc = z - mu
    var = jnp.mean(xc * xc, axis=0, keepdims=True)
    bn = xc * lax.rsqrt(var + EPS) * gamma + beta
    return jnp.where(bn >= 0.0, bn, alpha * bn)


def _tc_mid_body(p_ref, hp_ref, dis_ref, b_ref, g_ref, be_ref, a_ref, w_ref,
                 out_ref):
    agg = p_ref[0, :N] + p_ref[1, :N] + hp_ref[...]
    z = agg * dis_ref[...] + b_ref[...]
    act = _bn_prelu(z, g_ref[...], be_ref[...], a_ref[...])
    h = jnp.dot(act, w_ref[...], preferred_element_type=jnp.float32)
    out_ref[...] = h * dis_ref[...]


_tc_mid = pl.pallas_call(
    _tc_mid_body,
    out_shape=jax.ShapeDtypeStruct((N, D), jnp.float32),
)


def _tc_post_body(p_ref, hp_ref, dis_ref, b_ref, g_ref, be_ref, a_ref,
                  out_ref):
    agg = p_ref[0, :N] + p_ref[1, :N] + hp_ref[...]
    z = agg * dis_ref[...] + b_ref[...]
    out_ref[...] = _bn_prelu(z, g_ref[...], be_ref[...], a_ref[...])


_tc_post = pl.pallas_call(
    _tc_post_body,
    out_shape=jax.ShapeDtypeStruct((N, D), jnp.float32),
)


def kernel(x, edge_index, W1, b1, gamma1, beta1, a1, W2, b2, gamma2, beta2, a2):
    src = edge_index[0].astype(jnp.int32)
    dst = edge_index[1].astype(jnp.int32)

    npad = EPAD - E
    pad_pos = jnp.arange(npad, dtype=jnp.int32)
    src_p = jnp.concatenate([src, pad_pos % N])
    dst_p = jnp.concatenate([dst, N + (pad_pos % (ACC_N - N))])
    srcr = src_p.reshape(NW, CPT, CHUNK)
    dstr = dst_p.reshape(NW, CPT, CHUNK)

    h1 = _tc_mm1(x, W1)
    degp = _sc_deg(dstr)
    dis_row = _tc_deg(degp)
    dis = dis_row[0, :N].reshape(N, 1)

    b1r, g1r, be1r = (v.reshape(1, D) for v in (b1, gamma1, beta1))
    b2r, g2r, be2r = (v.reshape(1, D) for v in (b2, gamma2, beta2))
    a1r = a1.reshape(1, 1)
    a2r = a2.reshape(1, 1)

    hp1 = _tc_scale(h1, dis)
    p1 = _sc_agg(srcr, dstr, hp1)
    hp2 = _tc_mid(p1, hp1, dis, b1r, g1r, be1r, a1r, W2)
    p2 = _sc_agg(srcr, dstr, hp2)
    out = _tc_post(p2, hp2, dis, b2r, g2r, be2r, a2r)
    return out


# speedup vs baseline: 1.1948x; 1.0225x over previous
import functools

import jax
import jax.numpy as jnp
from jax import lax
from jax.experimental import pallas as pl
from jax.experimental.pallas import tpu as pltpu
from jax.experimental.pallas import tpu_sc as plsc

N = 10000
D = 128
E = 320000

NC = 2
NS = 16
NW = NC * NS

CHUNK = 64
CPT = 160
SG = 40
NSG = CPT // SG
NBUF = 4
DELTA = 3
NGRPS = SG // NBUF
EPT = CPT * CHUNK
EPAD = NW * EPT
ACC_N = 10240
ZRPT = ACC_N // NS
EPS = 1e-5

_mesh = plsc.VectorSubcoreMesh(core_axis_name="c", subcore_axis_name="s")


DEGW = 8


def _sc_deg_body(dst_hbm, deg_out, dstv, onesv, zbuf, acc, dsem):
    cid = lax.axis_index("c")
    sid = lax.axis_index("s")
    wid = cid * NS + sid

    pltpu.sync_copy(dst_hbm.at[wid], dstv)

    def _fill_ones(i, c):
        onesv[pl.ds(i * 16, 16)] = jnp.ones((16,), jnp.float32)
        return c
    lax.fori_loop(0, CHUNK // 16, _fill_ones, 0)

    @pl.when(sid == 0)
    def _init():
        def _z(i, c):
            zbuf[pl.ds(i * 16, 16)] = jnp.zeros((16,), jnp.float32)
            return c
        lax.fori_loop(0, ACC_N // 16, _z, 0)
        pltpu.sync_copy(zbuf, acc)

    plsc.subcore_barrier()

    def _chunk(j, c):
        pltpu.async_copy(onesv, acc.at[dstv.at[j]], dsem, add=True)

        @pl.when(j >= DEGW)
        def _w():
            pltpu.make_async_copy(onesv, acc.at[dstv.at[j - DEGW]],
                                  dsem).wait()
        return c
    lax.fori_loop(0, CPT, _chunk, 0)

    def _dw(j, c):
        pltpu.make_async_copy(onesv, acc.at[dstv.at[j]], dsem).wait()
        return c
    lax.fori_loop(CPT - DEGW, CPT, _dw, 0)

    plsc.subcore_barrier()

    @pl.when(sid == 0)
    def _out():
        pltpu.sync_copy(acc, deg_out.at[cid])


_sc_deg = pl.kernel(
    _sc_deg_body,
    out_type=jax.ShapeDtypeStruct((NC, ACC_N), jnp.float32),
    mesh=_mesh,
    scratch_types=[
        pltpu.VMEM((CPT, CHUNK), jnp.int32),
        pltpu.VMEM((CHUNK,), jnp.float32),
        pltpu.VMEM((ACC_N,), jnp.float32),
        pltpu.VMEM_SHARED((ACC_N,), jnp.float32),
        pltpu.SemaphoreType.DMA,
    ],
)


def _sc_agg_body(src_hbm, dst_hbm, hp_hbm, part_out, srcv, dstv, *scr):
    rows = scr[:NBUF]
    acc = scr[NBUF]
    gsem = scr[NBUF + 1:2 * NBUF + 1]
    ssem = scr[2 * NBUF + 1:]
    r0 = rows[0]
    cid = lax.axis_index("c")
    sid = lax.axis_index("s")
    wid = cid * NS + sid

    def _zr(r, c):
        def _zc(k, c2):
            r0[r, pl.ds(k * 16, 16)] = jnp.zeros((16,), jnp.float32)
            return c2
        return lax.fori_loop(0, D // 16, _zc, c)
    lax.fori_loop(0, CHUNK, _zr, 0)

    zbase = sid * ZRPT
    for k in range(ZRPT // CHUNK):
        pltpu.sync_copy(r0, acc.at[pl.ds(zbase + k * CHUNK, CHUNK)])

    plsc.subcore_barrier()

    def _gather(j, b):
        pltpu.async_copy(hp_hbm.at[srcv.at[j]], rows[b], gsem[b])

    def _gather_wait(j, b):
        pltpu.make_async_copy(hp_hbm.at[srcv.at[j]], rows[b], gsem[b]).wait()

    def _scatter(j, b):
        pltpu.async_copy(rows[b], acc.at[dstv.at[j]], ssem[b], add=True)

    def _scatter_wait(j, b):
        pltpu.make_async_copy(rows[b], acc.at[dstv.at[j]], ssem[b]).wait()

    def _stage(t, c):
        pltpu.sync_copy(src_hbm.at[wid, pl.ds(t * SG, SG)], srcv)
        pltpu.sync_copy(dst_hbm.at[wid, pl.ds(t * SG, SG)], dstv)

        def _group(g, c2):
            for b in range(NBUF):
                j = g * NBUF + b

                @pl.when(g > 0)
                def _drain(b=b, j=j):
                    _scatter_wait(j - NBUF, b)
                _gather(j, b)

                bs = (b - DELTA) % NBUF
                js = j - DELTA

                @pl.when(js >= 0)
                def _scat(js=js, bs=bs):
                    _gather_wait(js, bs)
                    _scatter(js, bs)
            return c2
        lax.fori_loop(0, NGRPS, _group, 0)

        for r in range(DELTA):
            js = SG - DELTA + r
            bs = js % NBUF
            _gather_wait(js, bs)
            _scatter(js, bs)
        for b in range(NBUF):
            _scatter_wait(SG - NBUF + b, b)
        return c
    lax.fori_loop(0, NSG, _stage, 0)

    plsc.subcore_barrier()

    obase = sid * ZRPT
    pltpu.sync_copy(acc.at[pl.ds(obase, ZRPT)],
                    part_out.at[cid, pl.ds(obase, ZRPT)])


_sc_agg = pl.kernel(
    _sc_agg_body,
    out_type=jax.ShapeDtypeStruct((NC, ACC_N, D), jnp.float32),
    mesh=_mesh,
    scratch_types=(
        [pltpu.VMEM((SG, CHUNK), jnp.int32),
         pltpu.VMEM((SG, CHUNK), jnp.int32)]
        + [pltpu.VMEM((CHUNK, D), jnp.float32) for _ in range(NBUF)]
        + [pltpu.VMEM_SHARED((ACC_N, D), jnp.float32)]
        + [pltpu.SemaphoreType.DMA for _ in range(2 * NBUF)]
    ),
)


def _tc_deg_body(degp_ref, dis_ref):
    deg = jnp.sum(degp_ref[...], axis=0, keepdims=True) + 1.0
    dis_ref[...] = lax.rsqrt(deg)


_tc_deg = pl.pallas_call(
    _tc_deg_body,
    out_shape=jax.ShapeDtypeStruct((1, ACC_N), jnp.float32),
)


def _tc_mm1_body(x_ref, w_ref, h_ref):
    h_ref[...] = jnp.dot(x_ref[...], w_ref[...],
                         preferred_element_type=jnp.float32)


_tc_mm1 = pl.pallas_call(
    _tc_mm1_body,
    out_shape=jax.ShapeDtypeStruct((N, D), jnp.float32),
)


def _tc_scale_body(h_ref, dis_ref, hp_ref):
    hp_ref[...] = h_ref[...] * dis_ref[...]


_tc_scale = pl.pallas_call(
    _tc_scale_body,
    out_shape=jax.ShapeDtypeStruct((N, D), jnp.float32),
)


def _bn_prelu(z, gamma, beta, alpha):
    mu = jnp.mean(z, axis=0, keepdims=True)
    xc = z - mu
    var = jnp.mean(xc * xc, axis=0, keepdims=True)
    bn = xc * lax.rsqrt(var + EPS) * gamma + beta
    return jnp.where(bn >= 0.0, bn, alpha * bn)


def _tc_mid_body(p_ref, hp_ref, dis_ref, b_ref, g_ref, be_ref, a_ref, w_ref,
                 out_ref):
    agg = p_ref[0, :N] + p_ref[1, :N] + hp_ref[...]
    z = agg * dis_ref[...] + b_ref[...]
    act = _bn_prelu(z, g_ref[...], be_ref[...], a_ref[...])
    h = jnp.dot(act, w_ref[...], preferred_element_type=jnp.float32)
    out_ref[...] = h * dis_ref[...]


_tc_mid = pl.pallas_call(
    _tc_mid_body,
    out_shape=jax.ShapeDtypeStruct((N, D), jnp.float32),
)


def _tc_post_body(p_ref, hp_ref, dis_ref, b_ref, g_ref, be_ref, a_ref,
                  out_ref):
    agg = p_ref[0, :N] + p_ref[1, :N] + hp_ref[...]
    z = agg * dis_ref[...] + b_ref[...]
    out_ref[...] = _bn_prelu(z, g_ref[...], be_ref[...], a_ref[...])


_tc_post = pl.pallas_call(
    _tc_post_body,
    out_shape=jax.ShapeDtypeStruct((N, D), jnp.float32),
)


def kernel(x, edge_index, W1, b1, gamma1, beta1, a1, W2, b2, gamma2, beta2, a2):
    src = edge_index[0].astype(jnp.int32)
    dst = edge_index[1].astype(jnp.int32)

    npad = EPAD - E
    pad_pos = jnp.arange(npad, dtype=jnp.int32)
    src_p = jnp.concatenate([src, pad_pos % N])
    dst_p = jnp.concatenate([dst, N + (pad_pos % (ACC_N - N))])
    srcr = src_p.reshape(NW, CPT, CHUNK)
    dstr = dst_p.reshape(NW, CPT, CHUNK)

    h1 = _tc_mm1(x, W1)
    degp = _sc_deg(dstr)
    dis_row = _tc_deg(degp)
    dis = dis_row[0, :N].reshape(N, 1)

    b1r, g1r, be1r = (v.reshape(1, D) for v in (b1, gamma1, beta1))
    b2r, g2r, be2r = (v.reshape(1, D) for v in (b2, gamma2, beta2))
    a1r = a1.reshape(1, 1)
    a2r = a2.reshape(1, 1)

    hp1 = _tc_scale(h1, dis)
    p1 = _sc_agg(srcr, dstr, hp1)
    hp2 = _tc_mid(p1, hp1, dis, b1r, g1r, be1r, a1r, W2)
    p2 = _sc_agg(srcr, dstr, hp2)
    out = _tc_post(p2, hp2, dis, b2r, g2r, be2r, a2r)
    return out


# speedup vs baseline: 1.1961x; 1.0010x over previous
import jax
import jax.numpy as jnp
from jax import lax
from jax.experimental import pallas as pl
from jax.experimental.pallas import tpu as pltpu
from jax.experimental.pallas import tpu_sc as plsc

N = 10000
D = 128
E = 320000

NC = 2
NS = 16
NW = NC * NS

CHUNK = 64
CPT = 160
SG = 40
NSG = CPT // SG
NBUF = 4
DELTA = 3
NGRPS = SG // NBUF
EPT = CPT * CHUNK
EPAD = NW * EPT
ACC_N = 10240
ZRPT = ACC_N // NS
EPS = 1e-5

_mesh = plsc.VectorSubcoreMesh(core_axis_name="c", subcore_axis_name="s")


DEGW = 8


def _sc_deg_body(dst_hbm, deg_out, dstv, onesv, zbuf, acc, dsem):
    cid = lax.axis_index("c")
    sid = lax.axis_index("s")
    wid = cid * NS + sid

    pltpu.sync_copy(dst_hbm.at[wid], dstv)

    def _fill_ones(i, c):
        onesv[pl.ds(i * 16, 16)] = jnp.ones((16,), jnp.float32)
        return c
    lax.fori_loop(0, CHUNK // 16, _fill_ones, 0)

    @pl.when(sid == 0)
    def _init():
        def _z(i, c):
            zbuf[pl.ds(i * 16, 16)] = jnp.zeros((16,), jnp.float32)
            return c
        lax.fori_loop(0, ACC_N // 16, _z, 0)
        pltpu.sync_copy(zbuf, acc)

    plsc.subcore_barrier()

    def _chunk(j, c):
        pltpu.async_copy(onesv, acc.at[dstv.at[j]], dsem, add=True)

        @pl.when(j >= DEGW)
        def _w():
            pltpu.make_async_copy(onesv, acc.at[dstv.at[j - DEGW]],
                                  dsem).wait()
        return c
    lax.fori_loop(0, CPT, _chunk, 0)

    def _dw(j, c):
        pltpu.make_async_copy(onesv, acc.at[dstv.at[j]], dsem).wait()
        return c
    lax.fori_loop(CPT - DEGW, CPT, _dw, 0)

    plsc.subcore_barrier()

    @pl.when(sid == 0)
    def _out():
        pltpu.sync_copy(acc, deg_out.at[cid])


_sc_deg = pl.kernel(
    _sc_deg_body,
    out_type=jax.ShapeDtypeStruct((NC, ACC_N), jnp.float32),
    mesh=_mesh,
    scratch_types=[
        pltpu.VMEM((CPT, CHUNK), jnp.int32),
        pltpu.VMEM((CHUNK,), jnp.float32),
        pltpu.VMEM((ACC_N,), jnp.float32),
        pltpu.VMEM_SHARED((ACC_N,), jnp.float32),
        pltpu.SemaphoreType.DMA,
    ],
)


def _sc_agg_body(src_hbm, dst_hbm, hp_hbm, part_out, srcv, dstv, *scr):
    rows = scr[:NBUF]
    acc = scr[NBUF]
    gsem = scr[NBUF + 1:2 * NBUF + 1]
    ssem = scr[2 * NBUF + 1:]
    r0 = rows[0]
    cid = lax.axis_index("c")
    sid = lax.axis_index("s")
    wid = cid * NS + sid

    def _zr(r, c):
        def _zc(k, c2):
            r0[r, pl.ds(k * 16, 16)] = jnp.zeros((16,), jnp.float32)
            return c2
        return lax.fori_loop(0, D // 16, _zc, c)
    lax.fori_loop(0, CHUNK, _zr, 0)

    zbase = sid * ZRPT
    for k in range(ZRPT // CHUNK):
        pltpu.sync_copy(r0, acc.at[pl.ds(zbase + k * CHUNK, CHUNK)])

    plsc.subcore_barrier()

    def _gather(j, b):
        pltpu.async_copy(hp_hbm.at[srcv.at[j]], rows[b], gsem[b])

    def _gather_wait(j, b):
        pltpu.make_async_copy(hp_hbm.at[srcv.at[j]], rows[b], gsem[b]).wait()

    def _scatter(j, b):
        pltpu.async_copy(rows[b], acc.at[dstv.at[j]], ssem[b], add=True)

    def _scatter_wait(j, b):
        pltpu.make_async_copy(rows[b], acc.at[dstv.at[j]], ssem[b]).wait()

    def _stage(t, c):
        pltpu.sync_copy(src_hbm.at[wid, pl.ds(t * SG, SG)], srcv)
        pltpu.sync_copy(dst_hbm.at[wid, pl.ds(t * SG, SG)], dstv)

        def _group(g, c2):
            for b in range(NBUF):
                j = g * NBUF + b

                @pl.when(g > 0)
                def _drain(b=b, j=j):
                    _scatter_wait(j - NBUF, b)
                _gather(j, b)

                bs = (b - DELTA) % NBUF
                js = j - DELTA

                @pl.when(js >= 0)
                def _scat(js=js, bs=bs):
                    _gather_wait(js, bs)
                    _scatter(js, bs)
            return c2
        lax.fori_loop(0, NGRPS, _group, 0)

        for r in range(DELTA):
            js = SG - DELTA + r
            bs = js % NBUF
            _gather_wait(js, bs)
            _scatter(js, bs)
        for b in range(NBUF):
            _scatter_wait(SG - NBUF + b, b)
        return c
    lax.fori_loop(0, NSG, _stage, 0)

    plsc.subcore_barrier()

    obase = sid * ZRPT
    pltpu.sync_copy(acc.at[pl.ds(obase, ZRPT)],
                    part_out.at[cid, pl.ds(obase, ZRPT)])


_sc_agg = pl.kernel(
    _sc_agg_body,
    out_type=jax.ShapeDtypeStruct((NC, ACC_N, D), jnp.float32),
    mesh=_mesh,
    scratch_types=(
        [pltpu.VMEM((SG, CHUNK), jnp.int32),
         pltpu.VMEM((SG, CHUNK), jnp.int32)]
        + [pltpu.VMEM((CHUNK, D), jnp.float32) for _ in range(NBUF)]
        + [pltpu.VMEM_SHARED((ACC_N, D), jnp.float32)]
        + [pltpu.SemaphoreType.DMA for _ in range(2 * NBUF)]
    ),
)


def _tc_predis_body(degp_ref, h_ref, dis_ref, hp_ref):
    deg = jnp.sum(degp_ref[...], axis=0, keepdims=True) + 1.0
    dis_col = jnp.transpose(lax.rsqrt(deg)[:, :N])
    dis_ref[...] = dis_col
    hp_ref[...] = h_ref[...] * dis_col


_tc_predis = pl.pallas_call(
    _tc_predis_body,
    out_shape=(jax.ShapeDtypeStruct((N, 1), jnp.float32),
               jax.ShapeDtypeStruct((N, D), jnp.float32)),
)


def _tc_mm1_body(x_ref, w_ref, h_ref):
    h_ref[...] = jnp.dot(x_ref[...], w_ref[...],
                         preferred_element_type=jnp.float32)


_tc_mm1 = pl.pallas_call(
    _tc_mm1_body,
    out_shape=jax.ShapeDtypeStruct((N, D), jnp.float32),
)


def _bn_prelu(z, gamma, beta, alpha):
    mu = jnp.mean(z, axis=0, keepdims=True)
    xc = z - mu
    var = jnp.mean(xc * xc, axis=0, keepdims=True)
    bn = xc * lax.rsqrt(var + EPS) * gamma + beta
    return jnp.where(bn >= 0.0, bn, alpha * bn)


def _tc_mid_body(p_ref, hp_ref, dis_ref, b_ref, g_ref, be_ref, a_ref, w_ref,
                 out_ref):
    agg = p_ref[0, :N] + p_ref[1, :N] + hp_ref[...]
    z = agg * dis_ref[...] + b_ref[...]
    act = _bn_prelu(z, g_ref[...], be_ref[...], a_ref[...])
    h = jnp.dot(act, w_ref[...], preferred_element_type=jnp.float32)
    out_ref[...] = h * dis_ref[...]


_tc_mid = pl.pallas_call(
    _tc_mid_body,
    out_shape=jax.ShapeDtypeStruct((N, D), jnp.float32),
)


def _tc_post_body(p_ref, hp_ref, dis_ref, b_ref, g_ref, be_ref, a_ref,
                  out_ref):
    agg = p_ref[0, :N] + p_ref[1, :N] + hp_ref[...]
    z = agg * dis_ref[...] + b_ref[...]
    out_ref[...] = _bn_prelu(z, g_ref[...], be_ref[...], a_ref[...])


_tc_post = pl.pallas_call(
    _tc_post_body,
    out_shape=jax.ShapeDtypeStruct((N, D), jnp.float32),
)


def kernel(x, edge_index, W1, b1, gamma1, beta1, a1, W2, b2, gamma2, beta2, a2):
    src = edge_index[0].astype(jnp.int32)
    dst = edge_index[1].astype(jnp.int32)

    npad = EPAD - E
    pad_pos = jnp.arange(npad, dtype=jnp.int32)
    src_p = jnp.concatenate([src, pad_pos % N])
    dst_p = jnp.concatenate([dst, N + (pad_pos % (ACC_N - N))])
    srcr = src_p.reshape(NW, CPT, CHUNK)
    dstr = dst_p.reshape(NW, CPT, CHUNK)

    h1 = _tc_mm1(x, W1)
    degp = _sc_deg(dstr)
    dis, hp1 = _tc_predis(degp, h1)

    b1r, g1r, be1r = (v.reshape(1, D) for v in (b1, gamma1, beta1))
    b2r, g2r, be2r = (v.reshape(1, D) for v in (b2, gamma2, beta2))
    a1r = a1.reshape(1, 1)
    a2r = a2.reshape(1, 1)

    p1 = _sc_agg(srcr, dstr, hp1)
    hp2 = _tc_mid(p1, hp1, dis, b1r, g1r, be1r, a1r, W2)
    p2 = _sc_agg(srcr, dstr, hp2)
    out = _tc_post(p2, hp2, dis, b2r, g2r, be2r, a2r)
    return out
